# baseline (speedup 1.0000x reference)
import functools

import jax
import jax.numpy as jnp
from jax import lax
from jax.experimental import pallas as pl
from jax.experimental.pallas import tpu as pltpu
from jax.experimental.pallas import tpu_sc as plsc

D = 64
NC = 2
NS = 16
NW = NC * NS
CHUNK = 128

_mesh = plsc.VectorSubcoreMesh(core_axis_name="c", subcore_axis_name="s")


def _make_sc_lookup(steps: int):
    @functools.partial(
        pl.kernel,
        mesh=_mesh,
        compiler_params=pltpu.CompilerParams(use_tc_tiling_on_sc=False),
        out_type=jax.ShapeDtypeStruct((NW, steps, CHUNK, D), jnp.float32),
        scratch_types=[
            pltpu.VMEM((steps, CHUNK), jnp.int32),
            pltpu.VMEM((CHUNK,), jnp.int32),
            pltpu.VMEM((CHUNK, D), jnp.float32),
            pltpu.SemaphoreType.DMA,
            pltpu.SemaphoreType.DMA,
        ],
    )
    def sc_lookup(ids_hbm, t2d_hbm, w_hbm, out_hbm, ids_v, dix_v, rows_v,
                  sem_a, sem_b):
        wid = lax.axis_index("s") * NC + lax.axis_index("c")
        pltpu.sync_copy(ids_hbm.at[wid], ids_v)

        def step(j, carry):
            pltpu.async_copy(t2d_hbm.at[ids_v.at[j]], dix_v, sem_a).wait()
            pltpu.async_copy(w_hbm.at[dix_v], rows_v, sem_b).wait()
            pltpu.sync_copy(rows_v, out_hbm.at[wid, j])
            return carry

        lax.fori_loop(0, steps, step, 0)

    return sc_lookup


def kernel(input_ids, attention_mask, tok2dict, embed_weight):
    B, L = input_ids.shape
    total = B * L
    assert total % (NW * CHUNK) == 0
    steps = total // (NW * CHUNK)
    ids = input_ids.reshape(NW, steps, CHUNK)
    out = _make_sc_lookup(steps)(ids, tok2dict, embed_weight)
    emb = out.reshape(B, L, D)
    return (emb, attention_mask.astype(jnp.int32))


# speedup vs baseline: 1.2017x; 1.2017x over previous
import functools

import jax
import jax.numpy as jnp
from jax import lax
from jax.experimental import pallas as pl
from jax.experimental.pallas import tpu as pltpu
from jax.experimental.pallas import tpu_sc as plsc

D = 64
NC = 2
NS = 16
NW = NC * NS
CHUNK = 1024

_mesh = plsc.VectorSubcoreMesh(core_axis_name="c", subcore_axis_name="s")


def _make_sc_lookup(steps: int):
    @functools.partial(
        pl.kernel,
        mesh=_mesh,
        compiler_params=pltpu.CompilerParams(use_tc_tiling_on_sc=False),
        out_type=jax.ShapeDtypeStruct((NW, steps, CHUNK, D), jnp.float32),
        scratch_types=[
            pltpu.VMEM((steps, CHUNK), jnp.int32),
            pltpu.VMEM((CHUNK,), jnp.int32),
            pltpu.VMEM((CHUNK, D), jnp.float32),
            pltpu.SemaphoreType.DMA,
            pltpu.SemaphoreType.DMA,
        ],
    )
    def sc_lookup(ids_hbm, t2d_hbm, w_hbm, out_hbm, ids_v, dix_v, rows_v,
                  sem_a, sem_b):
        wid = lax.axis_index("s") * NC + lax.axis_index("c")
        pltpu.sync_copy(ids_hbm.at[wid], ids_v)

        def step(j, carry):
            pltpu.async_copy(t2d_hbm.at[ids_v.at[j]], dix_v, sem_a).wait()
            pltpu.async_copy(w_hbm.at[dix_v], rows_v, sem_b).wait()
            pltpu.sync_copy(rows_v, out_hbm.at[wid, j])
            return carry

        lax.fori_loop(0, steps, step, 0)

    return sc_lookup


def kernel(input_ids, attention_mask, tok2dict, embed_weight):
    B, L = input_ids.shape
    total = B * L
    assert total % (NW * CHUNK) == 0
    steps = total // (NW * CHUNK)
    ids = input_ids.reshape(NW, steps, CHUNK)
    out = _make_sc_lookup(steps)(ids, tok2dict, embed_weight)
    emb = out.reshape(B, L, D)
    return (emb, attention_mask.astype(jnp.int32))


# speedup vs baseline: 1.2157x; 1.0117x over previous
import functools

import jax
import jax.numpy as jnp
from jax import lax
from jax.experimental import pallas as pl
from jax.experimental.pallas import tpu as pltpu
from jax.experimental.pallas import tpu_sc as plsc

D = 64
NC = 2
NS = 16
NW = NC * NS
CHUNK = 256
NBUF = 4

_mesh = plsc.VectorSubcoreMesh(core_axis_name="c", subcore_axis_name="s")


def _make_sc_lookup(steps: int):
    assert steps % NBUF == 0 and steps >= 3 * NBUF
    grp = steps // NBUF

    @functools.partial(
        pl.kernel,
        mesh=_mesh,
        compiler_params=pltpu.CompilerParams(use_tc_tiling_on_sc=False),
        out_type=jax.ShapeDtypeStruct((NW, steps, CHUNK, D), jnp.float32),
        scratch_types=[
            pltpu.VMEM((steps, CHUNK), jnp.int32),
            pltpu.VMEM((NBUF, CHUNK), jnp.int32),
            pltpu.VMEM((NBUF, CHUNK, D), jnp.float32),
        ] + [pltpu.SemaphoreType.DMA] * (3 * NBUF),
    )
    def sc_lookup(ids_hbm, t2d_hbm, w_hbm, out_hbm, ids_v, dix_v, rows_v,
                  *sems):
        dsem = sems[0:NBUF]
        rsem = sems[NBUF:2 * NBUF]
        ssem = sems[2 * NBUF:3 * NBUF]
        wid = lax.axis_index("s") * NC + lax.axis_index("c")
        pltpu.sync_copy(ids_hbm.at[wid], ids_v)

        def issue_d(j, slot):
            pltpu.async_copy(t2d_hbm.at[ids_v.at[j]], dix_v.at[slot],
                             dsem[slot])

        def wait_d(slot):
            pltpu.make_async_copy(t2d_hbm.at[pl.ds(0, CHUNK)],
                                  dix_v.at[slot], dsem[slot]).wait()

        def issue_r(slot):
            pltpu.async_copy(w_hbm.at[dix_v.at[slot]], rows_v.at[slot],
                             rsem[slot])

        def wait_r(slot):
            pltpu.make_async_copy(w_hbm.at[pl.ds(0, CHUNK)],
                                  rows_v.at[slot], rsem[slot]).wait()

        def issue_s(j, slot):
            pltpu.async_copy(rows_v.at[slot], out_hbm.at[wid, j], ssem[slot])

        def wait_s(slot):
            pltpu.make_async_copy(rows_v.at[slot], out_hbm.at[wid, 0],
                                  ssem[slot]).wait()

        for k in range(NBUF - 1):
            issue_d(k, k)

        for j in range(NBUF):
            if j >= 1:
                wait_r((j - 1) % NBUF)
                issue_s(j - 1, (j - 1) % NBUF)
                issue_d(j + NBUF - 1, (j - 1) % NBUF)
            else:
                issue_d(j + NBUF - 1, (j - 1) % NBUF)
            wait_d(j % NBUF)
            issue_r(j % NBUF)

        def group(g, carry):
            j0 = g * NBUF
            for b in range(NBUF):
                j = j0 + b
                pb = (b - 1) % NBUF
                wait_r(pb)
                issue_s(j - 1, pb)
                issue_d(j + NBUF - 1, pb)
                wait_s(b)
                wait_d(b)
                issue_r(b)
            return carry

        lax.fori_loop(1, grp - 1, group, 0)

        j0 = (grp - 1) * NBUF
        for b in range(NBUF):
            j = j0 + b
            pb = (b - 1) % NBUF
            wait_r(pb)
            issue_s(j - 1, pb)
            if j + NBUF - 1 < steps:
                issue_d(j + NBUF - 1, pb)
            wait_s(b)
            wait_d(b)
            issue_r(b)

        wait_r((steps - 1) % NBUF)
        issue_s(steps - 1, (steps - 1) % NBUF)
        for b in range(NBUF):
            wait_s(b)

    return sc_lookup


def kernel(input_ids, attention_mask, tok2dict, embed_weight):
    B, L = input_ids.shape
    total = B * L
    assert total % (NW * CHUNK) == 0
    steps = total // (NW * CHUNK)
    ids = input_ids.reshape(NW, steps, CHUNK)
    out = _make_sc_lookup(steps)(ids, tok2dict, embed_weight)
    emb = out.reshape(B, L, D)
    return (emb, attention_mask.astype(jnp.int32))
